# baseline (speedup 1.0000x reference)
import functools

import jax
import jax.numpy as jnp
from jax import lax
from jax.experimental import pallas as pl
from jax.experimental.pallas import tpu as pltpu
from jax.experimental.pallas import tpu_sc as plsc

_NW = 32
_NC = 2
_PBLK = 16384
_PSH = _PBLK.bit_length() - 1


def _pack_tc(wt):
    D, V = wt.shape
    ng = pl.cdiv(V, _PBLK)

    def body(x_ref, o_ref):
        t = x_ref[...].T
        o_ref[:, :D] = t[: _PBLK // 2]
        o_ref[:, D:] = t[_PBLK // 2 :]

    return pl.pallas_call(
        body,
        grid=(ng,),
        in_specs=[pl.BlockSpec((D, _PBLK), lambda i: (0, i))],
        out_specs=pl.BlockSpec((_PBLK // 2, 2 * D), lambda i: (i, 0)),
        out_shape=jax.ShapeDtypeStruct((ng * _PBLK // 2, 2 * D), jnp.float32),
    )(wt)


def _reflow_tc(flatp, y_prev, g, HC, B, H, D, BB=16384, WIN=1024):
    nb = B // BB
    hw = WIN // 2

    def body(*refs):
        x_ref, o_ref = refs[0], refs[-1]
        t = x_ref[...].T
        parts = []
        for w in range(BB // WIN):
            parts.append(t[:D, w * hw : (w + 1) * hw])
            parts.append(t[D:, w * hw : (w + 1) * hw])
        o_ref[...] = jnp.concatenate(parts, axis=1).reshape(1, D, BB)

    in_specs = [pl.BlockSpec((BB // 2, 2 * D), lambda h, j: (h * nb + j, 0))]
    args = [flatp]
    aliases = {}
    if y_prev is not None:
        in_specs.append(pl.BlockSpec(memory_space=pl.ANY))
        args.append(y_prev)
        aliases = {1: 0}

    return pl.pallas_call(
        body,
        grid=(HC, nb),
        in_specs=in_specs,
        out_specs=pl.BlockSpec((1, D, BB), lambda h, j: (g * HC + h, 0, j)),
        out_shape=jax.ShapeDtypeStruct((H, D, B), jnp.float32),
        input_output_aliases=aliases,
    )(*args)


def _pipeline(n_ch, nbuf, issue, wait, drain):
    n_main = (n_ch - nbuf) // nbuf

    for b in range(nbuf):
        issue(b, b)

    def block(jb, carry):
        jo = jb * nbuf
        for b in range(nbuf):
            wait(b)
            drain(jo + b, b)
            issue(jo + b + nbuf, b)
        return carry

    lax.fori_loop(0, n_main, block, 0)

    for j in range(n_main * nbuf, n_ch):
        b = j % nbuf
        wait(b)
        drain(j, b)
        if j + nbuf < n_ch:
            issue(j + nbuf, b)


def _make_gather_sc(n_rows, row0, VV, D, W, NBUF):
    per_w = n_rows // _NW
    mesh = plsc.VectorSubcoreMesh(core_axis_name="c", subcore_axis_name="s")

    @functools.partial(
        pl.kernel,
        mesh=mesh,
        compiler_params=pltpu.CompilerParams(
            use_tc_tiling_on_sc=False, needs_layout_passes=False
        ),
        out_type=jax.ShapeDtypeStruct((n_rows * W, D), jnp.float32),
        scratch_types=[
            pltpu.VMEM((per_w, W), jnp.int32),
            pltpu.VMEM((per_w, W), jnp.int32),
            pltpu.VMEM((NBUF, W, D), jnp.float32),
            pltpu.SemaphoreType.DMA,
        ],
    )
    def k(idx_hbm, tab_hbm, out_hbm, idx_v, idx_p, rows_v, sem):
        wid = lax.axis_index("s") * _NC + lax.axis_index("c")
        base = wid * per_w
        pltpu.sync_copy(idx_hbm.at[pl.ds(row0 + base, per_w)], idx_v)

        lane = jax.lax.iota(jnp.int32, 16)
        rofs = (lane & 1) * 4
        pos = lane >> 1

        def permute(j, carry):
            jg = j % 8
            row0 = j - jg + (jg >> 1)
            col0 = D * (jg & 1)
            rows = row0 + rofs
            for v in range(8):
                cols = col0 + 8 * v + pos
                r = plsc.load_gather(idx_v, [rows, cols])
                view = (
                    _PBLK * (r >> _PSH)
                    + 2 * (r & (_PBLK // 2 - 1))
                    + ((r >> (_PSH - 1)) & 1)
                )
                idx_p[j, pl.ds(16 * v, 16)] = view
            return carry

        lax.fori_loop(0, per_w, permute, 0)

        def issue(j, b):
            pltpu.async_copy(tab_hbm.at[idx_p.at[j]], rows_v.at[b], sem)

        def wait(b):
            pltpu.make_async_copy(
                tab_hbm.at[idx_p.at[0]], rows_v.at[b], sem
            ).wait()

        def drain(j, b):
            pltpu.sync_copy(
                rows_v.at[b], out_hbm.at[pl.ds((base + j) * W, W)]
            )

        _pipeline(per_w, NBUF, issue, wait, drain)

    return k


def kernel(input, weight_matrix):
    B, H = input.shape
    V, D = weight_matrix.shape
    N = B * H
    packed = _pack_tc(weight_matrix.T)
    table = packed.reshape(packed.shape[0] * 2, D)
    idx2 = input.T.reshape(N // 128, 128).astype(jnp.int32)
    G = 5
    rows_g = (N // 128) // G
    HC = H // G
    y = None
    for g in range(G):
        flat = _make_gather_sc(
            rows_g, g * rows_g, table.shape[0], D, 128, NBUF=8
        )(idx2, table)
        y = _reflow_tc(flat.reshape(rows_g * 64, 2 * D), y, g, HC, B, H, D)
    return y.transpose(2, 0, 1)


# speedup vs baseline: 1.0032x; 1.0032x over previous
import functools

import jax
import jax.numpy as jnp
from jax import lax
from jax.experimental import pallas as pl
from jax.experimental.pallas import tpu as pltpu
from jax.experimental.pallas import tpu_sc as plsc

_NW = 32
_NC = 2
_PBLK = 16384
_PSH = _PBLK.bit_length() - 1


def _pack_tc(wt):
    D, V = wt.shape
    ng = pl.cdiv(V, _PBLK)

    def body(x_ref, o_ref):
        t = x_ref[...].T
        o_ref[:, :D] = t[: _PBLK // 2]
        o_ref[:, D:] = t[_PBLK // 2 :]

    return pl.pallas_call(
        body,
        grid=(ng,),
        in_specs=[pl.BlockSpec((D, _PBLK), lambda i: (0, i))],
        out_specs=pl.BlockSpec((_PBLK // 2, 2 * D), lambda i: (i, 0)),
        out_shape=jax.ShapeDtypeStruct((ng * _PBLK // 2, 2 * D), jnp.float32),
    )(wt)


def _reflow_tc(flatp, y_prev, g, HC, B, H, D, BB=16384, WIN=512):
    nb = B // BB
    hw = WIN // 2

    def body(*refs):
        x_ref, o_ref = refs[0], refs[-1]
        t = x_ref[...].T
        parts = []
        for w in range(BB // WIN):
            parts.append(t[:D, w * hw : (w + 1) * hw])
            parts.append(t[D:, w * hw : (w + 1) * hw])
        o_ref[...] = jnp.concatenate(parts, axis=1).reshape(1, D, BB)

    in_specs = [pl.BlockSpec((BB // 2, 2 * D), lambda h, j: (h * nb + j, 0))]
    args = [flatp]
    aliases = {}
    if y_prev is not None:
        in_specs.append(pl.BlockSpec(memory_space=pl.ANY))
        args.append(y_prev)
        aliases = {1: 0}

    return pl.pallas_call(
        body,
        grid=(HC, nb),
        in_specs=in_specs,
        out_specs=pl.BlockSpec((1, D, BB), lambda h, j: (g * HC + h, 0, j)),
        out_shape=jax.ShapeDtypeStruct((H, D, B), jnp.float32),
        input_output_aliases=aliases,
    )(*args)


def _pipeline(n_ch, nbuf, issue, wait, drain):
    n_main = (n_ch - nbuf) // nbuf

    for b in range(nbuf):
        issue(b, b)

    def block(jb, carry):
        jo = jb * nbuf
        for b in range(nbuf):
            wait(b)
            drain(jo + b, b)
            issue(jo + b + nbuf, b)
        return carry

    lax.fori_loop(0, n_main, block, 0)

    for j in range(n_main * nbuf, n_ch):
        b = j % nbuf
        wait(b)
        drain(j, b)
        if j + nbuf < n_ch:
            issue(j + nbuf, b)


def _make_gather_sc(n_rows, row0, VV, D, W, NBUF):
    per_w = n_rows // _NW
    mesh = plsc.VectorSubcoreMesh(core_axis_name="c", subcore_axis_name="s")

    @functools.partial(
        pl.kernel,
        mesh=mesh,
        compiler_params=pltpu.CompilerParams(
            use_tc_tiling_on_sc=False, needs_layout_passes=False
        ),
        out_type=jax.ShapeDtypeStruct((n_rows * W, D), jnp.float32),
        scratch_types=[
            pltpu.VMEM((per_w, W), jnp.int32),
            pltpu.VMEM((per_w, W), jnp.int32),
            pltpu.VMEM((NBUF, W, D), jnp.float32),
            pltpu.SemaphoreType.DMA,
        ],
    )
    def k(idx_hbm, tab_hbm, out_hbm, idx_v, idx_p, rows_v, sem):
        wid = lax.axis_index("s") * _NC + lax.axis_index("c")
        base = wid * per_w
        pltpu.sync_copy(idx_hbm.at[pl.ds(row0 + base, per_w)], idx_v)

        lane = jax.lax.iota(jnp.int32, 16)
        rofs = (lane & 1) * 2
        pos = lane >> 1

        def permute(j, carry):
            jg = j % 4
            row0 = j - jg + (jg >> 1)
            col0 = D * (jg & 1)
            rows = row0 + rofs
            for v in range(8):
                cols = col0 + 8 * v + pos
                r = plsc.load_gather(idx_v, [rows, cols])
                view = (
                    _PBLK * (r >> _PSH)
                    + 2 * (r & (_PBLK // 2 - 1))
                    + ((r >> (_PSH - 1)) & 1)
                )
                idx_p[j, pl.ds(16 * v, 16)] = view
            return carry

        lax.fori_loop(0, per_w, permute, 0)

        def issue(j, b):
            pltpu.async_copy(tab_hbm.at[idx_p.at[j]], rows_v.at[b], sem)

        def wait(b):
            pltpu.make_async_copy(
                tab_hbm.at[idx_p.at[0]], rows_v.at[b], sem
            ).wait()

        def drain(j, b):
            pltpu.sync_copy(
                rows_v.at[b], out_hbm.at[pl.ds((base + j) * W, W)]
            )

        _pipeline(per_w, NBUF, issue, wait, drain)

    return k


def kernel(input, weight_matrix):
    B, H = input.shape
    V, D = weight_matrix.shape
    N = B * H
    packed = _pack_tc(weight_matrix.T)
    table = packed.reshape(packed.shape[0] * 2, D)
    idx2 = input.T.reshape(N // 128, 128).astype(jnp.int32)
    G = 2
    rows_g = (N // 128) // G
    HC = H // G
    y = None
    for g in range(G):
        flat = _make_gather_sc(
            rows_g, g * rows_g, table.shape[0], D, 128, NBUF=8
        )(idx2, table)
        y = _reflow_tc(flat.reshape(rows_g * 64, 2 * D), y, g, HC, B, H, D)
    return y.transpose(2, 0, 1)


# speedup vs baseline: 1.0379x; 1.0346x over previous
import functools

import jax
import jax.numpy as jnp
from jax import lax
from jax.experimental import pallas as pl
from jax.experimental.pallas import tpu as pltpu
from jax.experimental.pallas import tpu_sc as plsc

_NW = 32
_NC = 2
_PBLK = 32768
_PSH = _PBLK.bit_length() - 1


def _pack_tc(wt):
    D, V = wt.shape
    ng = pl.cdiv(V, _PBLK)

    def body(x_ref, o_ref):
        t = x_ref[...].T
        o_ref[:, :D] = t[: _PBLK // 2]
        o_ref[:, D:] = t[_PBLK // 2 :]

    return pl.pallas_call(
        body,
        grid=(ng,),
        in_specs=[pl.BlockSpec((D, _PBLK), lambda i: (0, i))],
        out_specs=pl.BlockSpec((_PBLK // 2, 2 * D), lambda i: (i, 0)),
        out_shape=jax.ShapeDtypeStruct((ng * _PBLK // 2, 2 * D), jnp.float32),
    )(wt)


def _reflow_tc(flatp, B, H, D, BB=16384, WIN=1024):
    nb = B // BB
    hw = WIN // 2

    def body(x_ref, o_ref):
        t = x_ref[...].T
        parts = []
        for w in range(BB // WIN):
            parts.append(t[:D, w * hw : (w + 1) * hw])
            parts.append(t[D:, w * hw : (w + 1) * hw])
        o_ref[...] = jnp.concatenate(parts, axis=1).reshape(1, D, BB)

    return pl.pallas_call(
        body,
        grid=(H, nb),
        in_specs=[pl.BlockSpec((BB // 2, 2 * D), lambda h, j: (h * nb + j, 0))],
        out_specs=pl.BlockSpec((1, D, BB), lambda h, j: (h, 0, j)),
        out_shape=jax.ShapeDtypeStruct((H, D, B), jnp.float32),
    )(flatp)


def _pipeline(n_ch, nbuf, issue, wait, drain):
    n_main = (n_ch - nbuf) // nbuf

    for b in range(nbuf):
        issue(b, b)

    def block(jb, carry):
        jo = jb * nbuf
        for b in range(nbuf):
            wait(b)
            drain(jo + b, b)
            issue(jo + b + nbuf, b)
        return carry

    lax.fori_loop(0, n_main, block, 0)

    for j in range(n_main * nbuf, n_ch):
        b = j % nbuf
        wait(b)
        drain(j, b)
        if j + nbuf < n_ch:
            issue(j + nbuf, b)


def _make_gather_sc(N, VV, D, W, NBUF):
    n_rows = N // W
    per_w = n_rows // _NW
    mesh = plsc.VectorSubcoreMesh(core_axis_name="c", subcore_axis_name="s")

    @functools.partial(
        pl.kernel,
        mesh=mesh,
        compiler_params=pltpu.CompilerParams(
            use_tc_tiling_on_sc=False, needs_layout_passes=False
        ),
        out_type=jax.ShapeDtypeStruct((N, D), jnp.float32),
        scratch_types=[
            pltpu.VMEM((per_w, W), jnp.int32),
            pltpu.VMEM((per_w, W), jnp.int32),
            pltpu.VMEM((NBUF, W, D), jnp.float32),
            pltpu.SemaphoreType.DMA,
        ],
    )
    def k(idx_hbm, tab_hbm, out_hbm, idx_v, idx_p, rows_v, sem):
        wid = lax.axis_index("s") * _NC + lax.axis_index("c")
        base = wid * per_w
        pltpu.sync_copy(idx_hbm.at[pl.ds(base, per_w)], idx_v)

        lane = jax.lax.iota(jnp.int32, 16)
        rofs = (lane & 1) * 4
        pos = lane >> 1

        def permute(j, carry):
            jg = j % 8
            row0 = j - jg + (jg >> 1)
            col0 = D * (jg & 1)
            rows = row0 + rofs
            for v in range(8):
                cols = col0 + 8 * v + pos
                r = plsc.load_gather(idx_v, [rows, cols])
                view = (
                    _PBLK * (r >> _PSH)
                    + 2 * (r & (_PBLK // 2 - 1))
                    + ((r >> (_PSH - 1)) & 1)
                )
                idx_p[j, pl.ds(16 * v, 16)] = view
            return carry

        lax.fori_loop(0, per_w, permute, 0)

        def issue(j, b):
            pltpu.async_copy(tab_hbm.at[idx_p.at[j]], rows_v.at[b], sem)

        def wait(b):
            pltpu.make_async_copy(
                tab_hbm.at[idx_p.at[0]], rows_v.at[b], sem
            ).wait()

        def drain(j, b):
            pltpu.sync_copy(
                rows_v.at[b], out_hbm.at[pl.ds((base + j) * W, W)]
            )

        _pipeline(per_w, NBUF, issue, wait, drain)

    return k


def kernel(input, weight_matrix):
    B, H = input.shape
    V, D = weight_matrix.shape
    N = B * H
    packed = _pack_tc(weight_matrix.T)
    table = packed.reshape(packed.shape[0] * 2, D)
    idx2 = input.T.reshape(N // 128, 128).astype(jnp.int32)
    flat = _make_gather_sc(N, table.shape[0], D, 128, NBUF=8)(idx2, table)
    y = _reflow_tc(flat.reshape(N // 2, 2 * D), B, H, D)
    return y.transpose(2, 0, 1)
